# baseline (speedup 1.0000x reference)
import functools

import jax
import jax.numpy as jnp
from jax import lax
from jax.experimental import pallas as pl
from jax.experimental.pallas import tpu as pltpu
from jax.experimental.pallas import tpu_sc as plsc

HID = 128
RBF = 32
L = 16
NTILES = 32
FPT = HID // NTILES
CH = 4000
G = 2048
NBLK = 1280
DMAX = 6.4


def _ssp(v):
    return jnp.maximum(v, 0.0) + jnp.log(1.0 + jnp.exp(-jnp.abs(v))) - 0.5


def _wid():
    return lax.axis_index("s") * 2 + lax.axis_index("c")


def _geom_body(e, n, ept, n_pad, ei_hbm, xtf_hbm, d2_out, deg_out,
               x0v, x1v, x2v, ivb, jvb, d2b, degb):
    wid = _wid()
    base = wid * ept
    pltpu.sync_copy(xtf_hbm.at[pl.ds(0, n)], x0v)
    pltpu.sync_copy(xtf_hbm.at[pl.ds(n, n)], x1v)
    pltpu.sync_copy(xtf_hbm.at[pl.ds(2 * n, n)], x2v)
    pltpu.sync_copy(ei_hbm.at[pl.ds(base, ept)], ivb)
    pltpu.sync_copy(ei_hbm.at[pl.ds(e + base, ept)], jvb)

    zf = jnp.zeros((L,), jnp.float32)

    @plsc.parallel_loop(0, n_pad, step=L, unroll=2)
    def _zero(off):
        degb[pl.ds(off, L)] = zf

    ones = jnp.full((L,), 1.0, jnp.float32)

    @plsc.parallel_loop(0, ept, step=L, unroll=1)
    def _edges(off):
        iv = ivb[pl.ds(off, L)]
        jv = jvb[pl.ds(off, L)]
        d0 = plsc.load_gather(x0v, [iv]) - plsc.load_gather(x0v, [jv])
        d1 = plsc.load_gather(x1v, [iv]) - plsc.load_gather(x1v, [jv])
        d2 = plsc.load_gather(x2v, [iv]) - plsc.load_gather(x2v, [jv])
        d2b[pl.ds(off, L)] = d0 * d0 + d1 * d1 + d2 * d2
        plsc.addupdate_scatter(degb, [iv], ones)

    pltpu.sync_copy(d2b, d2_out.at[pl.ds(base, ept)])
    pltpu.sync_copy(degb, deg_out.at[wid])


def _scatter_body(e, nch, n_pad, ei_hbm, p_hbm, wlutT_hbm, hT_hbm,
                  aggrT_out, hT4, acc, wlut4,
                  ibuf0, ibuf1, pbuf0, pbuf1, sem0, sem1):
    wid = _wid()
    f0 = wid * FPT
    p0 = wid * (FPT // 2)
    pltpu.sync_copy(hT_hbm.at[pl.ds(p0, FPT // 2)], hT4)
    pltpu.sync_copy(wlutT_hbm.at[pl.ds(p0, FPT // 2)], wlut4)

    zf = jnp.zeros((L,), jnp.float32)

    @plsc.parallel_loop(0, n_pad, step=L, unroll=2)
    def _zero(off):
        for f in range(FPT):
            acc[f, pl.ds(off, L)] = zf

    fvs = [jnp.full((L,), f, jnp.int32) for f in range(FPT)]
    pvs = [jnp.full((L,), p, jnp.int32) for p in range(FPT // 2)]
    m14 = jnp.full((L,), 16383, jnp.int32)
    s14 = jnp.full((L,), 14, jnp.int32)
    s16 = jnp.full((L,), 16, jnp.int32)
    mhi = jnp.full((L,), -65536, jnp.int32)
    bufs = ((ibuf0, pbuf0, sem0), (ibuf1, pbuf1, sem1))

    def _start(c, slot):
        ib, pb, sem = bufs[slot]
        eoff = c * CH
        pltpu.async_copy(ei_hbm.at[pl.ds(eoff, CH)], ib, sem)
        pltpu.async_copy(p_hbm.at[pl.ds(eoff, CH)], pb, sem)

    def _drain(slot):
        ib, pb, sem = bufs[slot]
        pltpu.make_async_copy(ei_hbm.at[pl.ds(0, CH)], ib, sem).wait()
        pltpu.make_async_copy(p_hbm.at[pl.ds(0, CH)], pb, sem).wait()

    _start(0, 0)

    def outer(g, carry):
        for b in range(2):
            c = g * 2 + b
            _drain(b)

            @pl.when(c + 1 < nch)
            def _():
                _start(c + 1, 1 - b)

            ib, pb, _sem = bufs[b]

            @plsc.parallel_loop(0, CH, step=L, unroll=2)
            def _inner(off):
                iv = ib[pl.ds(off, L)]
                pv = pb[pl.ds(off, L)]
                jv = lax.bitwise_and(pv, m14)
                kv = lax.shift_right_logical(pv, s14)
                for p in range(FPT // 2):
                    wp = plsc.load_gather(wlut4, [pvs[p], kv])
                    hp = plsc.load_gather(hT4, [pvs[p], jv])
                    wlo = plsc.bitcast(lax.shift_left(wp, s16), jnp.float32)
                    hlo = plsc.bitcast(lax.shift_left(hp, s16), jnp.float32)
                    whi = plsc.bitcast(lax.bitwise_and(wp, mhi), jnp.float32)
                    hhi = plsc.bitcast(lax.bitwise_and(hp, mhi), jnp.float32)
                    plsc.addupdate_scatter(acc, [fvs[p], iv], wlo * hlo)
                    plsc.addupdate_scatter(acc, [fvs[p + 2], iv], whi * hhi)
        return carry
    lax.fori_loop(0, nch // 2, outer, 0)

    half = FPT // 2
    pltpu.sync_copy(acc.at[pl.ds(0, half)], aggrT_out.at[pl.ds(p0, half)])
    pltpu.sync_copy(acc.at[pl.ds(half, half)],
                    aggrT_out.at[pl.ds(HID // 2 + p0, half)])


def _lut_body(fW1_ref, fb1_ref, fW2_ref, fb2_ref, out_ref):
    d = lax.broadcasted_iota(jnp.int32, (G, 1), 0).astype(
        jnp.float32) * (DMAX / (G - 1))
    centers = lax.broadcasted_iota(jnp.int32, (1, RBF), 1).astype(
        jnp.float32) * (5.0 / (RBF - 1))
    t = d - centers
    rbf = jnp.exp(-10.0 * t * t)
    a = jnp.dot(rbf, fW1_ref[...], preferred_element_type=jnp.float32)
    a = _ssp(a + fb1_ref[...])
    w = jnp.dot(a, fW2_ref[...], preferred_element_type=jnp.float32)
    out_ref[...] = _pack_pairs((w + fb2_ref[...]).T)


def _quant_body(d2_ref, j_ref, p_ref):
    d = jnp.sqrt(d2_ref[...] + 1e-12)
    k = (d * ((G - 1) / DMAX) + 0.5).astype(jnp.int32)
    p_ref[...] = j_ref[...] + jnp.minimum(k, G - 1) * 16384


def _pack_pairs(m):
    bi = lax.bitcast_convert_type(m, jnp.int32) + 32768
    half = m.shape[0] // 2
    lo = lax.shift_right_logical(bi[:half, :], 16)
    hi = lax.bitwise_and(bi[half:, :], jnp.int32(-65536))
    return lax.bitwise_or(lo, hi)


def _transpose_body(x_ref, o_ref):
    o_ref[...] = _pack_pairs(x_ref[...].T)


def _update_body(aggrT_ref, degP_ref, uW1_ref, ub1_ref, uW2_ref, ub2_ref,
                 out_ref):
    deg = jnp.sum(degP_ref[...], axis=0, keepdims=True)
    an = aggrT_ref[...] / jnp.maximum(deg, 1.0)
    a = an.T
    h1 = _ssp(jnp.dot(a, uW1_ref[...], preferred_element_type=jnp.float32)
              + ub1_ref[...])
    out_ref[...] = (jnp.dot(h1, uW2_ref[...], preferred_element_type=jnp.float32)
                    + ub2_ref[...])


def kernel(h, x, edge_index, fW1, fb1, fW2, fb2, uW1, ub1, uW2, ub2):
    n = h.shape[0]
    e = edge_index.shape[1]

    n_pad = -(-n // NBLK) * NBLK
    assert e % (NTILES * L) == 0 and e % CH == 0 and (e // CH) % 2 == 0
    ept = e // NTILES
    nch = e // CH

    ei = edge_index.astype(jnp.int32).reshape(-1)

    mesh = plsc.VectorSubcoreMesh(core_axis_name="c", subcore_axis_name="s")
    f32 = jnp.float32

    geom = pl.kernel(
        functools.partial(_geom_body, e, n, ept, n_pad),
        out_type=(jax.ShapeDtypeStruct((e,), f32),
                  jax.ShapeDtypeStruct((NTILES, n_pad), f32)),
        mesh=mesh,
        compiler_params=pltpu.CompilerParams(needs_layout_passes=False),
        scratch_types=[
            pltpu.VMEM((n,), f32),
            pltpu.VMEM((n,), f32),
            pltpu.VMEM((n,), f32),
            pltpu.VMEM((ept,), jnp.int32),
            pltpu.VMEM((ept,), jnp.int32),
            pltpu.VMEM((ept,), f32),
            pltpu.VMEM((n_pad,), f32),
        ],
    )
    d2, degP = geom(ei, x.T.reshape(-1))

    wlutT = pl.pallas_call(
        _lut_body,
        out_shape=jax.ShapeDtypeStruct((HID // 2, G), jnp.int32),
    )(fW1, fb1.reshape(1, HID), fW2, fb2.reshape(1, HID))

    pk = pl.pallas_call(
        _quant_body,
        out_shape=jax.ShapeDtypeStruct((e,), jnp.int32),
    )(d2, ei[e:])

    hT = pl.pallas_call(
        _transpose_body,
        grid=(n_pad // NBLK,),
        in_specs=[pl.BlockSpec((NBLK, HID), lambda b: (b, 0))],
        out_specs=pl.BlockSpec((HID // 2, NBLK), lambda b: (0, b)),
        out_shape=jax.ShapeDtypeStruct((HID // 2, n_pad), jnp.int32),
    )(h)

    scat = pl.kernel(
        functools.partial(_scatter_body, e, nch, n_pad),
        out_type=jax.ShapeDtypeStruct((HID, n_pad), f32),
        mesh=mesh,
        compiler_params=pltpu.CompilerParams(needs_layout_passes=False),
        scratch_types=[
            pltpu.VMEM((FPT // 2, n_pad), jnp.int32),
            pltpu.VMEM((FPT, n_pad), f32),
            pltpu.VMEM((FPT // 2, G), jnp.int32),
            pltpu.VMEM((CH,), jnp.int32),
            pltpu.VMEM((CH,), jnp.int32),
            pltpu.VMEM((CH,), jnp.int32),
            pltpu.VMEM((CH,), jnp.int32),
            pltpu.SemaphoreType.DMA,
            pltpu.SemaphoreType.DMA,
        ],
    )
    aggrT = scat(ei, pk, wlutT, hT)

    h_new = pl.pallas_call(
        _update_body,
        grid=(n_pad // NBLK,),
        in_specs=[
            pl.BlockSpec((HID, NBLK), lambda b: (0, b)),
            pl.BlockSpec((NTILES, NBLK), lambda b: (0, b)),
            pl.BlockSpec((HID, HID), lambda b: (0, 0)),
            pl.BlockSpec((1, HID), lambda b: (0, 0)),
            pl.BlockSpec((HID, HID), lambda b: (0, 0)),
            pl.BlockSpec((1, HID), lambda b: (0, 0)),
        ],
        out_specs=pl.BlockSpec((NBLK, HID), lambda b: (b, 0)),
        out_shape=jax.ShapeDtypeStruct((n, HID), f32),
    )(aggrT, degP, uW1, ub1.reshape(1, HID), uW2, ub2.reshape(1, HID))

    return (h_new, x)


# speedup vs baseline: 1.0043x; 1.0043x over previous
import functools

import jax
import jax.numpy as jnp
from jax import lax
from jax.experimental import pallas as pl
from jax.experimental.pallas import tpu as pltpu
from jax.experimental.pallas import tpu_sc as plsc

HID = 128
RBF = 32
L = 16
NTILES = 32
FPT = HID // NTILES
CH = 2000
G = 2048
NBLK = 1280
DMAX = 6.4


def _ssp(v):
    return jnp.maximum(v, 0.0) + jnp.log(1.0 + jnp.exp(-jnp.abs(v))) - 0.5


def _wid():
    return lax.axis_index("s") * 2 + lax.axis_index("c")


def _geom_body(e, n, ept, n_pad, ei_hbm, xtf_hbm, d2_out, deg_out,
               x0v, x1v, x2v, ivb, jvb, d2b, degb):
    wid = _wid()
    base = wid * ept
    pltpu.sync_copy(xtf_hbm.at[pl.ds(0, n)], x0v)
    pltpu.sync_copy(xtf_hbm.at[pl.ds(n, n)], x1v)
    pltpu.sync_copy(xtf_hbm.at[pl.ds(2 * n, n)], x2v)
    pltpu.sync_copy(ei_hbm.at[pl.ds(base, ept)], ivb)
    pltpu.sync_copy(ei_hbm.at[pl.ds(e + base, ept)], jvb)

    zf = jnp.zeros((L,), jnp.float32)

    @plsc.parallel_loop(0, n_pad, step=L, unroll=2)
    def _zero(off):
        degb[pl.ds(off, L)] = zf

    ones = jnp.full((L,), 1.0, jnp.float32)

    @plsc.parallel_loop(0, ept, step=L, unroll=1)
    def _edges(off):
        iv = ivb[pl.ds(off, L)]
        jv = jvb[pl.ds(off, L)]
        d0 = plsc.load_gather(x0v, [iv]) - plsc.load_gather(x0v, [jv])
        d1 = plsc.load_gather(x1v, [iv]) - plsc.load_gather(x1v, [jv])
        d2 = plsc.load_gather(x2v, [iv]) - plsc.load_gather(x2v, [jv])
        d2b[pl.ds(off, L)] = d0 * d0 + d1 * d1 + d2 * d2
        plsc.addupdate_scatter(degb, [iv], ones)

    pltpu.sync_copy(d2b, d2_out.at[pl.ds(base, ept)])
    pltpu.sync_copy(degb, deg_out.at[wid])


def _scatter_body(e, nch, n_pad, ei_hbm, p_hbm, wlutT_hbm, hT_hbm,
                  aggrT_out, hT4, acc, wlut4,
                  ibuf0, ibuf1, pbuf0, pbuf1, sem0, sem1):
    wid = _wid()
    f0 = wid * FPT
    p0 = wid * (FPT // 2)
    pltpu.sync_copy(hT_hbm.at[pl.ds(p0, FPT // 2)], hT4)
    pltpu.sync_copy(wlutT_hbm.at[pl.ds(p0, FPT // 2)], wlut4)

    zf = jnp.zeros((L,), jnp.float32)

    @plsc.parallel_loop(0, n_pad, step=L, unroll=2)
    def _zero(off):
        for f in range(FPT):
            acc[f, pl.ds(off, L)] = zf

    fvs = [jnp.full((L,), f, jnp.int32) for f in range(FPT)]
    pvs = [jnp.full((L,), p, jnp.int32) for p in range(FPT // 2)]
    m14 = jnp.full((L,), 16383, jnp.int32)
    s14 = jnp.full((L,), 14, jnp.int32)
    s16 = jnp.full((L,), 16, jnp.int32)
    mhi = jnp.full((L,), -65536, jnp.int32)
    bufs = ((ibuf0, pbuf0, sem0), (ibuf1, pbuf1, sem1))

    def _start(c, slot):
        ib, pb, sem = bufs[slot]
        eoff = c * CH
        pltpu.async_copy(ei_hbm.at[pl.ds(eoff, CH)], ib, sem)
        pltpu.async_copy(p_hbm.at[pl.ds(eoff, CH)], pb, sem)

    def _drain(slot):
        ib, pb, sem = bufs[slot]
        pltpu.make_async_copy(ei_hbm.at[pl.ds(0, CH)], ib, sem).wait()
        pltpu.make_async_copy(p_hbm.at[pl.ds(0, CH)], pb, sem).wait()

    _start(0, 0)

    def outer(g, carry):
        for b in range(2):
            c = g * 2 + b
            _drain(b)

            @pl.when(c + 1 < nch)
            def _():
                _start(c + 1, 1 - b)

            ib, pb, _sem = bufs[b]

            @plsc.parallel_loop(0, CH, step=L, unroll=1)
            def _inner(off):
                iv = ib[pl.ds(off, L)]
                pv = pb[pl.ds(off, L)]
                jv = lax.bitwise_and(pv, m14)
                kv = lax.shift_right_logical(pv, s14)
                for p in range(FPT // 2):
                    wp = plsc.load_gather(wlut4, [pvs[p], kv])
                    hp = plsc.load_gather(hT4, [pvs[p], jv])
                    wlo = plsc.bitcast(lax.shift_left(wp, s16), jnp.float32)
                    hlo = plsc.bitcast(lax.shift_left(hp, s16), jnp.float32)
                    whi = plsc.bitcast(lax.bitwise_and(wp, mhi), jnp.float32)
                    hhi = plsc.bitcast(lax.bitwise_and(hp, mhi), jnp.float32)
                    plsc.addupdate_scatter(acc, [fvs[p], iv], wlo * hlo)
                    plsc.addupdate_scatter(acc, [fvs[p + 2], iv], whi * hhi)
        return carry
    lax.fori_loop(0, nch // 2, outer, 0)

    half = FPT // 2
    pltpu.sync_copy(acc.at[pl.ds(0, half)], aggrT_out.at[pl.ds(p0, half)])
    pltpu.sync_copy(acc.at[pl.ds(half, half)],
                    aggrT_out.at[pl.ds(HID // 2 + p0, half)])


def _lut_body(fW1_ref, fb1_ref, fW2_ref, fb2_ref, out_ref):
    d = lax.broadcasted_iota(jnp.int32, (G, 1), 0).astype(
        jnp.float32) * (DMAX / (G - 1))
    centers = lax.broadcasted_iota(jnp.int32, (1, RBF), 1).astype(
        jnp.float32) * (5.0 / (RBF - 1))
    t = d - centers
    rbf = jnp.exp(-10.0 * t * t)
    a = jnp.dot(rbf, fW1_ref[...], preferred_element_type=jnp.float32)
    a = _ssp(a + fb1_ref[...])
    w = jnp.dot(a, fW2_ref[...], preferred_element_type=jnp.float32)
    out_ref[...] = _pack_pairs((w + fb2_ref[...]).T)


def _quant_body(d2_ref, j_ref, p_ref):
    d = jnp.sqrt(d2_ref[...] + 1e-12)
    k = (d * ((G - 1) / DMAX) + 0.5).astype(jnp.int32)
    p_ref[...] = j_ref[...] + jnp.minimum(k, G - 1) * 16384


def _pack_pairs(m):
    bi = lax.bitcast_convert_type(m, jnp.int32) + 32768
    half = m.shape[0] // 2
    lo = lax.shift_right_logical(bi[:half, :], 16)
    hi = lax.bitwise_and(bi[half:, :], jnp.int32(-65536))
    return lax.bitwise_or(lo, hi)


def _transpose_body(x_ref, o_ref):
    o_ref[...] = _pack_pairs(x_ref[...].T)


def _update_body(aggrT_ref, degP_ref, uW1_ref, ub1_ref, uW2_ref, ub2_ref,
                 out_ref):
    deg = jnp.sum(degP_ref[...], axis=0, keepdims=True)
    an = aggrT_ref[...] / jnp.maximum(deg, 1.0)
    a = an.T
    h1 = _ssp(jnp.dot(a, uW1_ref[...], preferred_element_type=jnp.float32)
              + ub1_ref[...])
    out_ref[...] = (jnp.dot(h1, uW2_ref[...], preferred_element_type=jnp.float32)
                    + ub2_ref[...])


def kernel(h, x, edge_index, fW1, fb1, fW2, fb2, uW1, ub1, uW2, ub2):
    n = h.shape[0]
    e = edge_index.shape[1]

    n_pad = -(-n // NBLK) * NBLK
    assert e % (NTILES * L) == 0 and e % CH == 0 and (e // CH) % 2 == 0
    ept = e // NTILES
    nch = e // CH

    ei = edge_index.astype(jnp.int32).reshape(-1)

    mesh = plsc.VectorSubcoreMesh(core_axis_name="c", subcore_axis_name="s")
    f32 = jnp.float32

    geom = pl.kernel(
        functools.partial(_geom_body, e, n, ept, n_pad),
        out_type=(jax.ShapeDtypeStruct((e,), f32),
                  jax.ShapeDtypeStruct((NTILES, n_pad), f32)),
        mesh=mesh,
        compiler_params=pltpu.CompilerParams(needs_layout_passes=False),
        scratch_types=[
            pltpu.VMEM((n,), f32),
            pltpu.VMEM((n,), f32),
            pltpu.VMEM((n,), f32),
            pltpu.VMEM((ept,), jnp.int32),
            pltpu.VMEM((ept,), jnp.int32),
            pltpu.VMEM((ept,), f32),
            pltpu.VMEM((n_pad,), f32),
        ],
    )
    d2, degP = geom(ei, x.T.reshape(-1))

    wlutT = pl.pallas_call(
        _lut_body,
        out_shape=jax.ShapeDtypeStruct((HID // 2, G), jnp.int32),
    )(fW1, fb1.reshape(1, HID), fW2, fb2.reshape(1, HID))

    pk = pl.pallas_call(
        _quant_body,
        out_shape=jax.ShapeDtypeStruct((e,), jnp.int32),
    )(d2, ei[e:])

    hT = pl.pallas_call(
        _transpose_body,
        grid=(n_pad // NBLK,),
        in_specs=[pl.BlockSpec((NBLK, HID), lambda b: (b, 0))],
        out_specs=pl.BlockSpec((HID // 2, NBLK), lambda b: (0, b)),
        out_shape=jax.ShapeDtypeStruct((HID // 2, n_pad), jnp.int32),
    )(h)

    scat = pl.kernel(
        functools.partial(_scatter_body, e, nch, n_pad),
        out_type=jax.ShapeDtypeStruct((HID, n_pad), f32),
        mesh=mesh,
        compiler_params=pltpu.CompilerParams(needs_layout_passes=False),
        scratch_types=[
            pltpu.VMEM((FPT // 2, n_pad), jnp.int32),
            pltpu.VMEM((FPT, n_pad), f32),
            pltpu.VMEM((FPT // 2, G), jnp.int32),
            pltpu.VMEM((CH,), jnp.int32),
            pltpu.VMEM((CH,), jnp.int32),
            pltpu.VMEM((CH,), jnp.int32),
            pltpu.VMEM((CH,), jnp.int32),
            pltpu.SemaphoreType.DMA,
            pltpu.SemaphoreType.DMA,
        ],
    )
    aggrT = scat(ei, pk, wlutT, hT)

    h_new = pl.pallas_call(
        _update_body,
        grid=(n_pad // NBLK,),
        in_specs=[
            pl.BlockSpec((HID, NBLK), lambda b: (0, b)),
            pl.BlockSpec((NTILES, NBLK), lambda b: (0, b)),
            pl.BlockSpec((HID, HID), lambda b: (0, 0)),
            pl.BlockSpec((1, HID), lambda b: (0, 0)),
            pl.BlockSpec((HID, HID), lambda b: (0, 0)),
            pl.BlockSpec((1, HID), lambda b: (0, 0)),
        ],
        out_specs=pl.BlockSpec((NBLK, HID), lambda b: (b, 0)),
        out_shape=jax.ShapeDtypeStruct((n, HID), f32),
    )(aggrT, degP, uW1, ub1.reshape(1, HID), uW2, ub2.reshape(1, HID))

    return (h_new, x)


# speedup vs baseline: 1.0164x; 1.0121x over previous
import functools

import jax
import jax.numpy as jnp
from jax import lax
from jax.experimental import pallas as pl
from jax.experimental.pallas import tpu as pltpu
from jax.experimental.pallas import tpu_sc as plsc

HID = 128
RBF = 32
L = 16
NTILES = 32
FPT = HID // NTILES
CH = 4000
G = 2048
NBLK = 1280
DMAX = 6.4


def _ssp(v):
    return jnp.maximum(v, 0.0) + jnp.log(1.0 + jnp.exp(-jnp.abs(v))) - 0.5


def _wid():
    return lax.axis_index("s") * 2 + lax.axis_index("c")


def _geom_body(e, n, ept, n_pad, ei_hbm, xtf_hbm, d2_out, deg_out,
               x0v, x1v, x2v, ivb, jvb, d2b, degb):
    wid = _wid()
    base = wid * ept
    pltpu.sync_copy(xtf_hbm.at[pl.ds(0, n)], x0v)
    pltpu.sync_copy(xtf_hbm.at[pl.ds(n, n)], x1v)
    pltpu.sync_copy(xtf_hbm.at[pl.ds(2 * n, n)], x2v)
    pltpu.sync_copy(ei_hbm.at[pl.ds(base, ept)], ivb)
    pltpu.sync_copy(ei_hbm.at[pl.ds(e + base, ept)], jvb)

    zf = jnp.zeros((L,), jnp.float32)

    @plsc.parallel_loop(0, n_pad, step=L, unroll=2)
    def _zero(off):
        degb[pl.ds(off, L)] = zf

    ones = jnp.full((L,), 1.0, jnp.float32)

    @plsc.parallel_loop(0, ept, step=L, unroll=1)
    def _edges(off):
        iv = ivb[pl.ds(off, L)]
        jv = jvb[pl.ds(off, L)]
        d0 = plsc.load_gather(x0v, [iv]) - plsc.load_gather(x0v, [jv])
        d1 = plsc.load_gather(x1v, [iv]) - plsc.load_gather(x1v, [jv])
        d2 = plsc.load_gather(x2v, [iv]) - plsc.load_gather(x2v, [jv])
        d2b[pl.ds(off, L)] = d0 * d0 + d1 * d1 + d2 * d2
        plsc.addupdate_scatter(degb, [iv], ones)

    pltpu.sync_copy(d2b, d2_out.at[pl.ds(base, ept)])
    pltpu.sync_copy(degb, deg_out.at[wid])


def _scatter_body(e, nch, n_pad, ei_hbm, p_hbm, wlutT_hbm, hT_hbm,
                  aggrT_out, hT4, acc, wlut4,
                  ibuf0, ibuf1, pbuf0, pbuf1, sem0, sem1):
    wid = _wid()
    f0 = wid * FPT
    p0 = wid * (FPT // 2)
    pltpu.sync_copy(hT_hbm.at[pl.ds(p0, FPT // 2)], hT4)
    pltpu.sync_copy(wlutT_hbm.at[pl.ds(p0, FPT // 2)], wlut4)

    zf = jnp.zeros((L,), jnp.float32)

    @plsc.parallel_loop(0, n_pad, step=L, unroll=2)
    def _zero(off):
        for f in range(FPT):
            acc[f, pl.ds(off, L)] = zf

    fvs = [jnp.full((L,), f, jnp.int32) for f in range(FPT)]
    pvs = [jnp.full((L,), p, jnp.int32) for p in range(FPT // 2)]
    m14 = jnp.full((L,), 16383, jnp.int32)
    s14 = jnp.full((L,), 14, jnp.int32)
    s16 = jnp.full((L,), 16, jnp.int32)
    mhi = jnp.full((L,), -65536, jnp.int32)
    bufs = ((ibuf0, pbuf0, sem0), (ibuf1, pbuf1, sem1))

    def _start(c, slot):
        ib, pb, sem = bufs[slot]
        eoff = c * CH
        pltpu.async_copy(ei_hbm.at[pl.ds(eoff, CH)], ib, sem)
        pltpu.async_copy(p_hbm.at[pl.ds(eoff, CH)], pb, sem)

    def _drain(slot):
        ib, pb, sem = bufs[slot]
        pltpu.make_async_copy(ei_hbm.at[pl.ds(0, CH)], ib, sem).wait()
        pltpu.make_async_copy(p_hbm.at[pl.ds(0, CH)], pb, sem).wait()

    _start(0, 0)

    def outer(g, carry):
        for b in range(2):
            c = g * 2 + b
            _drain(b)

            @pl.when(c + 1 < nch)
            def _():
                _start(c + 1, 1 - b)

            ib, pb, _sem = bufs[b]

            @plsc.parallel_loop(0, CH, step=L, unroll=1)
            def _inner(off):
                iv = ib[pl.ds(off, L)]
                pv = pb[pl.ds(off, L)]
                jv = lax.bitwise_and(pv, m14)
                kv = lax.shift_right_logical(pv, s14)
                for p in range(FPT // 2):
                    wp = plsc.load_gather(wlut4, [pvs[p], kv])
                    hp = plsc.load_gather(hT4, [pvs[p], jv])
                    wlo = plsc.bitcast(lax.shift_left(wp, s16), jnp.float32)
                    hlo = plsc.bitcast(lax.shift_left(hp, s16), jnp.float32)
                    whi = plsc.bitcast(lax.bitwise_and(wp, mhi), jnp.float32)
                    hhi = plsc.bitcast(lax.bitwise_and(hp, mhi), jnp.float32)
                    plsc.addupdate_scatter(acc, [fvs[p], iv], wlo * hlo)
                    plsc.addupdate_scatter(acc, [fvs[p + 2], iv], whi * hhi)
        return carry
    lax.fori_loop(0, nch // 2, outer, 0)

    half = FPT // 2
    pltpu.sync_copy(acc.at[pl.ds(0, half)], aggrT_out.at[pl.ds(p0, half)])
    pltpu.sync_copy(acc.at[pl.ds(half, half)],
                    aggrT_out.at[pl.ds(HID // 2 + p0, half)])


def _lut_body(fW1_ref, fb1_ref, fW2_ref, fb2_ref, out_ref):
    d = lax.broadcasted_iota(jnp.int32, (G, 1), 0).astype(
        jnp.float32) * (DMAX / (G - 1))
    centers = lax.broadcasted_iota(jnp.int32, (1, RBF), 1).astype(
        jnp.float32) * (5.0 / (RBF - 1))
    t = d - centers
    rbf = jnp.exp(-10.0 * t * t)
    a = jnp.dot(rbf, fW1_ref[...], preferred_element_type=jnp.float32)
    a = _ssp(a + fb1_ref[...])
    w = jnp.dot(a, fW2_ref[...], preferred_element_type=jnp.float32)
    out_ref[...] = _pack_pairs((w + fb2_ref[...]).T)


def _quant_body(d2_ref, j_ref, p_ref):
    d = jnp.sqrt(d2_ref[...] + 1e-12)
    k = (d * ((G - 1) / DMAX) + 0.5).astype(jnp.int32)
    p_ref[...] = j_ref[...] + jnp.minimum(k, G - 1) * 16384


def _pack_pairs(m):
    bi = lax.bitcast_convert_type(m, jnp.int32) + 32768
    half = m.shape[0] // 2
    lo = lax.shift_right_logical(bi[:half, :], 16)
    hi = lax.bitwise_and(bi[half:, :], jnp.int32(-65536))
    return lax.bitwise_or(lo, hi)


def _transpose_body(x_ref, o_ref):
    o_ref[...] = _pack_pairs(x_ref[...].T)


def _update_body(aggrT_ref, degP_ref, uW1_ref, ub1_ref, uW2_ref, ub2_ref,
                 out_ref):
    deg = jnp.sum(degP_ref[...], axis=0, keepdims=True)
    an = aggrT_ref[...] / jnp.maximum(deg, 1.0)
    a = an.T
    h1 = _ssp(jnp.dot(a, uW1_ref[...], preferred_element_type=jnp.float32)
              + ub1_ref[...])
    out_ref[...] = (jnp.dot(h1, uW2_ref[...], preferred_element_type=jnp.float32)
                    + ub2_ref[...])


def kernel(h, x, edge_index, fW1, fb1, fW2, fb2, uW1, ub1, uW2, ub2):
    n = h.shape[0]
    e = edge_index.shape[1]

    n_pad = -(-n // NBLK) * NBLK
    assert e % (NTILES * L) == 0 and e % CH == 0 and (e // CH) % 2 == 0
    ept = e // NTILES
    nch = e // CH

    ei = edge_index.astype(jnp.int32).reshape(-1)

    mesh = plsc.VectorSubcoreMesh(core_axis_name="c", subcore_axis_name="s")
    f32 = jnp.float32

    geom = pl.kernel(
        functools.partial(_geom_body, e, n, ept, n_pad),
        out_type=(jax.ShapeDtypeStruct((e,), f32),
                  jax.ShapeDtypeStruct((NTILES, n_pad), f32)),
        mesh=mesh,
        compiler_params=pltpu.CompilerParams(needs_layout_passes=False),
        scratch_types=[
            pltpu.VMEM((n,), f32),
            pltpu.VMEM((n,), f32),
            pltpu.VMEM((n,), f32),
            pltpu.VMEM((ept,), jnp.int32),
            pltpu.VMEM((ept,), jnp.int32),
            pltpu.VMEM((ept,), f32),
            pltpu.VMEM((n_pad,), f32),
        ],
    )
    d2, degP = geom(ei, x.T.reshape(-1))

    wlutT = pl.pallas_call(
        _lut_body,
        out_shape=jax.ShapeDtypeStruct((HID // 2, G), jnp.int32),
    )(fW1, fb1.reshape(1, HID), fW2, fb2.reshape(1, HID))

    pk = pl.pallas_call(
        _quant_body,
        out_shape=jax.ShapeDtypeStruct((e,), jnp.int32),
    )(d2, ei[e:])

    hT = pl.pallas_call(
        _transpose_body,
        grid=(n_pad // NBLK,),
        in_specs=[pl.BlockSpec((NBLK, HID), lambda b: (b, 0))],
        out_specs=pl.BlockSpec((HID // 2, NBLK), lambda b: (0, b)),
        out_shape=jax.ShapeDtypeStruct((HID // 2, n_pad), jnp.int32),
    )(h)

    scat = pl.kernel(
        functools.partial(_scatter_body, e, nch, n_pad),
        out_type=jax.ShapeDtypeStruct((HID, n_pad), f32),
        mesh=mesh,
        compiler_params=pltpu.CompilerParams(needs_layout_passes=False),
        scratch_types=[
            pltpu.VMEM((FPT // 2, n_pad), jnp.int32),
            pltpu.VMEM((FPT, n_pad), f32),
            pltpu.VMEM((FPT // 2, G), jnp.int32),
            pltpu.VMEM((CH,), jnp.int32),
            pltpu.VMEM((CH,), jnp.int32),
            pltpu.VMEM((CH,), jnp.int32),
            pltpu.VMEM((CH,), jnp.int32),
            pltpu.SemaphoreType.DMA,
            pltpu.SemaphoreType.DMA,
        ],
    )
    aggrT = scat(ei, pk, wlutT, hT)

    h_new = pl.pallas_call(
        _update_body,
        grid=(n_pad // NBLK,),
        in_specs=[
            pl.BlockSpec((HID, NBLK), lambda b: (0, b)),
            pl.BlockSpec((NTILES, NBLK), lambda b: (0, b)),
            pl.BlockSpec((HID, HID), lambda b: (0, 0)),
            pl.BlockSpec((1, HID), lambda b: (0, 0)),
            pl.BlockSpec((HID, HID), lambda b: (0, 0)),
            pl.BlockSpec((1, HID), lambda b: (0, 0)),
        ],
        out_specs=pl.BlockSpec((NBLK, HID), lambda b: (b, 0)),
        out_shape=jax.ShapeDtypeStruct((n, HID), f32),
    )(aggrT, degP, uW1, ub1.reshape(1, HID), uW2, ub2.reshape(1, HID))

    return (h_new, x)
